# baseline (speedup 1.0000x reference)
import functools

import jax
import jax.numpy as jnp
import numpy as np
from jax import lax
from jax.experimental import pallas as pl
from jax.experimental.pallas import tpu as pltpu
from jax.experimental.pallas import tpu_sc as plsc

D = 64
L_SEQ = 200
NC = 2
NS = 16
NW = NC * NS
LANES = 16


def _sinusoidal_pe_np(length, d_model):
    pos = np.arange(length, dtype=np.float32)[:, None]
    div = np.exp(
        np.arange(0, d_model, 2, dtype=np.float32) * (-np.log(10000.0) / d_model)
    )
    pe = np.zeros((length, d_model), dtype=np.float32)
    pe[:, 0::2] = np.sin(pos * div)
    pe[:, 1::2] = np.cos(pos * div)
    return pe


@functools.partial(jax.jit, static_argnames=("n_rows",))
def _embed(idx_flat, token_table, pe, n_rows):
    rows_per_w = n_rows // NW
    seqs_per_w = rows_per_w // L_SEQ
    mesh = plsc.VectorSubcoreMesh(core_axis_name="c", subcore_axis_name="s")

    @functools.partial(
        pl.kernel,
        out_type=jax.ShapeDtypeStruct((n_rows, D), jnp.float32),
        mesh=mesh,
        scratch_types=[
            pltpu.VMEM((rows_per_w,), jnp.int32),
            pltpu.VMEM((L_SEQ, D), jnp.float32),
            pltpu.VMEM((2, L_SEQ, D), jnp.float32),
            pltpu.SemaphoreType.DMA,
            pltpu.SemaphoreType.DMA,
        ],
        compiler_params=pltpu.CompilerParams(use_tc_tiling_on_sc=False),
    )
    def k(table_hbm, idx_hbm, pe_hbm, out_hbm, idx_v, pe_v, rows_v, sem0, sem1):
        wid = lax.axis_index("s") * NC + lax.axis_index("c")
        base = wid * rows_per_w
        pltpu.sync_copy(idx_hbm.at[pl.ds(base, rows_per_w)], idx_v)
        pltpu.sync_copy(pe_hbm, pe_v)
        sems = (sem0, sem1)

        def start(s, b):
            pltpu.async_copy(
                table_hbm.at[idx_v.at[pl.ds(s * L_SEQ, L_SEQ)]],
                rows_v.at[b],
                sems[b],
            )

        def wait(s, b):
            pltpu.make_async_copy(
                table_hbm.at[idx_v.at[pl.ds(s * L_SEQ, L_SEQ)]],
                rows_v.at[b],
                sems[b],
            ).wait()

        def process(s, b):
            wait(s, b)
            rbuf = rows_v.at[b]

            def add_pe(r, _):
                for d in range(D // LANES):
                    sl = pl.ds(d * LANES, LANES)
                    rbuf[r, sl] = rbuf[r, sl] + pe_v[r, sl]
                return 0

            lax.fori_loop(0, L_SEQ, add_pe, 0)
            pltpu.sync_copy(rbuf, out_hbm.at[pl.ds(base + s * L_SEQ, L_SEQ)])

        start(0, 0)
        start(1, 1)

        def body(g, _):
            s = 2 * g
            process(s, 0)
            start(s + 2, 0)
            process(s + 1, 1)
            start(s + 3, 1)
            return 0

        lax.fori_loop(0, seqs_per_w // 2 - 1, body, 0)
        process(seqs_per_w - 2, 0)
        process(seqs_per_w - 1, 1)

    return k(token_table, idx_flat, pe)


def kernel(sequence, token_table):
    B, L = sequence.shape
    idx_flat = sequence.reshape(-1).astype(jnp.int32)
    pe = jnp.asarray(_sinusoidal_pe_np(L, token_table.shape[1]))
    out = _embed(idx_flat, token_table, pe, B * L)
    return out.reshape(B, L, token_table.shape[1])
